# baseline (speedup 1.0000x reference)
import functools

import jax
import jax.numpy as jnp
from jax import lax
from jax.experimental import pallas as pl
from jax.experimental.pallas import tpu as pltpu
from jax.experimental.pallas import tpu_sc as plsc

N = 10000
E = 320000
D_IN = 128
HID = 128
K = 64
OUT = 64

NC = 2
NS = 16
NW = NC * NS
CHUNK = 125
NCHUNK = E // (NW * CHUNK)
NBUF = 4
N_PAD = 10240
RPT = N_PAD // NS
STG = 320


def _stage_indices_and_zero(row3, col3, row_v, col_v, znk, acc, w, sid):
    pltpu.sync_copy(row3.at[w], row_v)
    pltpu.sync_copy(col3.at[w], col_v)
    sl = pl.ds(sid * RPT, RPT)
    pltpu.sync_copy(znk.at[sl], acc.at[sl])
    return sl


def _scatter_pipeline(tab, row_v, col_v, vals, acc, gsem, ssem, deg_args):
    for b in range(NBUF):
        pltpu.async_copy(tab.at[row_v.at[b]], vals.at[b], gsem.at[b])

    def quad(i, carry):
        for b in range(NBUF):
            j = NBUF * i + b
            pltpu.make_async_copy(tab.at[row_v.at[j]], vals.at[b],
                                  gsem.at[b]).wait()
            pltpu.async_copy(vals.at[b], acc.at[col_v.at[j]], ssem.at[b],
                             add=True)
            if deg_args is not None:
                ones_v, deg, dsem = deg_args
                pltpu.async_copy(ones_v, deg.at[col_v.at[j]], dsem,
                                 add=True)

            @pl.when(j + NBUF < NCHUNK)
            def _():
                pltpu.make_async_copy(vals.at[b], acc.at[col_v.at[j]],
                                      ssem.at[b]).wait()
                pltpu.async_copy(tab.at[row_v.at[j + NBUF]], vals.at[b],
                                 gsem.at[b])
            if deg_args is not None:
                ones_v, deg, dsem = deg_args

                @pl.when(j >= NBUF)
                def _():
                    pltpu.make_async_copy(ones_v, deg.at[col_v.at[j]],
                                          dsem).wait()
        return carry

    lax.fori_loop(0, NCHUNK // NBUF, quad, 0)

    for b in range(NBUF):
        j = NCHUNK - NBUF + b
        pltpu.make_async_copy(vals.at[b], acc.at[col_v.at[j]],
                              ssem.at[b]).wait()
        if deg_args is not None:
            ones_v, deg, dsem = deg_args
            pltpu.make_async_copy(ones_v, deg.at[col_v.at[j]], dsem).wait()


def _segsum_deg_body(tab, row3, col3, znk, zn, ones_h, out, outdeg,
                     row_v, col_v, vals, ones_v, acc, deg, gsem, ssem, dsem):
    cid = lax.axis_index("c")
    sid = lax.axis_index("s")
    w = cid * NS + sid
    sl = _stage_indices_and_zero(row3, col3, row_v, col_v, znk, acc, w, sid)
    pltpu.sync_copy(ones_h, ones_v)

    @pl.when(sid == 0)
    def _():
        pltpu.sync_copy(zn, deg)
    plsc.subcore_barrier()

    _scatter_pipeline(tab, row_v, col_v, vals, acc, gsem, ssem,
                      (ones_v, deg, dsem))

    plsc.subcore_barrier()
    pltpu.sync_copy(acc.at[sl], out.at[cid, sl])

    @pl.when(sid == 0)
    def _():
        pltpu.sync_copy(deg, outdeg.at[cid])


def _segsum_num_body(tab, row3, col3, znk, outnum,
                     row_v, col_v, vals, num_v, acc, gsem, ssem):
    cid = lax.axis_index("c")
    sid = lax.axis_index("s")
    w = cid * NS + sid
    _stage_indices_and_zero(row3, col3, row_v, col_v, znk, acc, w, sid)
    plsc.subcore_barrier()

    _scatter_pipeline(tab, row_v, col_v, vals, acc, gsem, ssem, None)

    plsc.subcore_barrier()

    base = sid * RPT
    total = jnp.zeros((16,), jnp.float32)
    off = 0
    for sz in [120, 120, 120, 120, 120, 40]:
        pltpu.sync_copy(acc.at[pl.ds(base + off, sz)],
                        vals.at[0, pl.ds(0, sz)])
        pltpu.sync_copy(tab.at[pl.ds(base + off, sz)],
                        vals.at[1, pl.ds(0, sz)])

        def rowbody(r, c):
            c0, c1, c2, c3 = c
            c0 = c0 + vals[0, r, pl.ds(0, 16)] * vals[1, r, pl.ds(0, 16)]
            c1 = c1 + vals[0, r, pl.ds(16, 16)] * vals[1, r, pl.ds(16, 16)]
            c2 = c2 + vals[0, r, pl.ds(32, 16)] * vals[1, r, pl.ds(32, 16)]
            c3 = c3 + vals[0, r, pl.ds(48, 16)] * vals[1, r, pl.ds(48, 16)]
            return (c0, c1, c2, c3)

        z16 = jnp.zeros((16,), jnp.float32)
        c0, c1, c2, c3 = lax.fori_loop(0, sz, rowbody, (z16, z16, z16, z16))
        total = total + c0 + c1 + c2 + c3
        off += sz
    num_v[...] = total
    pltpu.sync_copy(num_v, outnum.at[cid, sid])


@functools.cache
def _make_segsum_deg():
    mesh = plsc.VectorSubcoreMesh(core_axis_name="c", subcore_axis_name="s",
                                  num_cores=NC, num_subcores=NS)
    return pl.kernel(
        _segsum_deg_body,
        out_type=[jax.ShapeDtypeStruct((NC, N_PAD, K), jnp.float32),
                  jax.ShapeDtypeStruct((NC, N_PAD, 16), jnp.float32)],
        mesh=mesh,
        scratch_types=[
            pltpu.VMEM((NCHUNK, CHUNK), jnp.int32),
            pltpu.VMEM((NCHUNK, CHUNK), jnp.int32),
            pltpu.VMEM((NBUF, CHUNK, K), jnp.float32),
            pltpu.VMEM((CHUNK, 16), jnp.float32),
            pltpu.VMEM_SHARED((N_PAD, K), jnp.float32),
            pltpu.VMEM_SHARED((N_PAD, 16), jnp.float32),
            pltpu.SemaphoreType.DMA((NBUF,)),
            pltpu.SemaphoreType.DMA((NBUF,)),
            pltpu.SemaphoreType.DMA,
        ],
        compiler_params=pltpu.CompilerParams(use_tc_tiling_on_sc=False))


@functools.cache
def _make_segsum_num():
    mesh = plsc.VectorSubcoreMesh(core_axis_name="c", subcore_axis_name="s",
                                  num_cores=NC, num_subcores=NS)
    return pl.kernel(
        _segsum_num_body,
        out_type=jax.ShapeDtypeStruct((NC, NS, 16), jnp.float32),
        mesh=mesh,
        scratch_types=[
            pltpu.VMEM((NCHUNK, CHUNK), jnp.int32),
            pltpu.VMEM((NCHUNK, CHUNK), jnp.int32),
            pltpu.VMEM((NBUF, CHUNK, K), jnp.float32),
            pltpu.VMEM((16,), jnp.float32),
            pltpu.VMEM_SHARED((N_PAD, K), jnp.float32),
            pltpu.SemaphoreType.DMA((NBUF,)),
            pltpu.SemaphoreType.DMA((NBUF,)),
        ],
        compiler_params=pltpu.CompilerParams(use_tc_tiling_on_sc=False))


SMR = N_PAD // NW


def _softmax_sc_body(xap, beig, s_out, a_ref, b_ref, o_ref, beig_v):
    cid = lax.axis_index("c")
    sid = lax.axis_index("s")
    w = cid * NS + sid
    base = w * SMR
    pltpu.sync_copy(xap.at[0, pl.ds(base, SMR)], a_ref)
    pltpu.sync_copy(xap.at[1, pl.ds(base, SMR)], b_ref)
    pltpu.sync_copy(beig, beig_v)
    iota = lax.broadcasted_iota(jnp.int32, (16,), 0)

    def grp(g, carry):
        ri = g * 16 + iota

        def colpass(c, ssum):
            ci = jnp.full((16,), c, jnp.int32)
            av = plsc.load_gather(a_ref, [ri, ci])
            bv = plsc.load_gather(b_ref, [ri, ci])
            be = plsc.load_gather(beig_v, [ci])
            e = jnp.exp(av + bv + be)
            plsc.store_scatter(o_ref, [ri, ci], e)
            return ssum + e

        ssum = lax.fori_loop(0, K, colpass, jnp.zeros((16,), jnp.float32))
        rinv = 1.0 / ssum

        def colscale(c, cc):
            ci = jnp.full((16,), c, jnp.int32)
            ev = plsc.load_gather(o_ref, [ri, ci])
            plsc.store_scatter(o_ref, [ri, ci], ev * rinv)
            return cc

        lax.fori_loop(0, K, colscale, 0)
        return carry

    lax.fori_loop(0, SMR // 16, grp, 0)
    pltpu.sync_copy(o_ref, s_out.at[pl.ds(base, SMR)])


@functools.cache
def _make_softmax_sc():
    mesh = plsc.VectorSubcoreMesh(core_axis_name="c", subcore_axis_name="s",
                                  num_cores=NC, num_subcores=NS)
    return pl.kernel(
        _softmax_sc_body,
        out_type=jax.ShapeDtypeStruct((N_PAD, K), jnp.float32),
        mesh=mesh,
        scratch_types=[
            pltpu.VMEM((SMR, K), jnp.float32),
            pltpu.VMEM((SMR, K), jnp.float32),
            pltpu.VMEM((SMR, K), jnp.float32),
            pltpu.VMEM((K,), jnp.float32),
        ],
        compiler_params=pltpu.CompilerParams(use_tc_tiling_on_sc=False,
                                             needs_layout_passes=False))


BM = 2000


def _mm_body(x, wx, bx, xx_ref):
    xx_ref[...] = (jnp.dot(x[...], wx[...], preferred_element_type=jnp.float32)
                   + bx[...])


def _dense_body(s_in, xx, dg, wf, bf, z_ref, ss_ref, den_ref, ss_acc,
                den_acc):
    i = pl.program_id(0)

    @pl.when(i == 0)
    def _():
        ss_acc[...] = jnp.zeros_like(ss_acc)
        den_acc[0, 0] = 0.0

    sb = s_in[...]
    deg = dg[0][:, 0:1] + dg[1][:, 0:1]
    ssq = jnp.sum(sb * sb, axis=1, keepdims=True)
    den_acc[0, 0] += jnp.sum(deg * ssq)
    ss_acc[...] += lax.dot_general(sb, sb, (((0,), (0,)), ((), ())),
                                   preferred_element_type=jnp.float32)

    z = (jnp.dot(xx[...], wf[:HID, :], preferred_element_type=jnp.float32)
         + jnp.dot(sb, wf[HID:, :], preferred_element_type=jnp.float32)
         + bf[...])
    m = jnp.max(z, axis=1, keepdims=True)
    z_ref[...] = z - m - jnp.log(jnp.sum(jnp.exp(z - m), axis=1,
                                         keepdims=True))

    @pl.when(i == pl.num_programs(0) - 1)
    def _():
        ss_ref[...] = ss_acc[...]
        den_ref[...] = jnp.full((1, 1), den_acc[0, 0], jnp.float32)


def _loss_body(ss, den, nump, loss_ref):
    SS = ss[...]
    fro = jnp.sqrt(jnp.sum(SS * SS))
    r = lax.broadcasted_iota(jnp.int32, (K, K), 0)
    c = lax.broadcasted_iota(jnp.int32, (K, K), 1)
    eye = jnp.where(r == c, 1.0, 0.0).astype(jnp.float32)
    T = SS / (fro + 1e-10) - eye / jnp.sqrt(jnp.float32(K))
    ortho = jnp.sqrt(jnp.sum(T * T))
    num = jnp.sum(nump[...])
    pump = -(num / (jnp.sum(den[...]) + 1e-10))
    loss_ref[...] = jnp.full((1, 1), pump + ortho, jnp.float32)


def kernel(x, edge_index, W_eig, b_eig, W_x, b_x, W_f, b_f):
    row3 = edge_index[0].reshape(NW, NCHUNK, CHUNK)
    col3 = edge_index[1].reshape(NW, NCHUNK, CHUNK)
    znk = jnp.zeros((N_PAD, K), jnp.float32)
    zn = jnp.zeros((N_PAD, 16), jnp.float32)
    ones_c = jnp.zeros((CHUNK, 16), jnp.float32).at[:, 0].set(1.0)

    grid = (N // BM,)

    xx = pl.pallas_call(
        _mm_body,
        grid=grid,
        in_specs=[
            pl.BlockSpec((BM, D_IN), lambda i: (i, 0)),
            pl.BlockSpec((D_IN, HID), lambda i: (0, 0)),
            pl.BlockSpec((1, HID), lambda i: (0, 0)),
        ],
        out_specs=pl.BlockSpec((BM, HID), lambda i: (i, 0)),
        out_shape=jax.ShapeDtypeStruct((N, HID), jnp.float32),
    )(x, W_x, b_x.reshape(1, HID))

    xap, degp = _make_segsum_deg()(W_eig, row3, col3, znk, zn, ones_c)

    s = _make_softmax_sc()(xap, b_eig)

    nump = _make_segsum_num()(s, row3, col3, znk)

    z, ss, den = pl.pallas_call(
        _dense_body,
        grid=grid,
        in_specs=[
            pl.BlockSpec((BM, K), lambda i: (i, 0)),
            pl.BlockSpec((BM, HID), lambda i: (i, 0)),
            pl.BlockSpec((NC, BM, 16), lambda i: (0, i, 0)),
            pl.BlockSpec((HID + K, OUT), lambda i: (0, 0)),
            pl.BlockSpec((1, OUT), lambda i: (0, 0)),
        ],
        out_specs=[
            pl.BlockSpec((BM, OUT), lambda i: (i, 0)),
            pl.BlockSpec((K, K), lambda i: (0, 0)),
            pl.BlockSpec((1, 1), lambda i: (0, 0)),
        ],
        out_shape=[
            jax.ShapeDtypeStruct((N, OUT), jnp.float32),
            jax.ShapeDtypeStruct((K, K), jnp.float32),
            jax.ShapeDtypeStruct((1, 1), jnp.float32),
        ],
        scratch_shapes=[
            pltpu.VMEM((K, K), jnp.float32),
            pltpu.SMEM((1, 1), jnp.float32),
        ],
    )(s, xx, degp, W_f, b_f.reshape(1, OUT))

    loss = pl.pallas_call(
        _loss_body,
        grid=(1,),
        in_specs=[
            pl.BlockSpec((K, K), lambda i: (0, 0)),
            pl.BlockSpec((1, 1), lambda i: (0, 0)),
            pl.BlockSpec((NC, NS, 16), lambda i: (0, 0, 0)),
        ],
        out_specs=pl.BlockSpec((1, 1), lambda i: (0, 0)),
        out_shape=jax.ShapeDtypeStruct((1, 1), jnp.float32),
    )(ss, den, nump)

    return z, loss[0, 0]


# speedup vs baseline: 1.3199x; 1.3199x over previous
import functools

import jax
import jax.numpy as jnp
from jax import lax
from jax.experimental import pallas as pl
from jax.experimental.pallas import tpu as pltpu
from jax.experimental.pallas import tpu_sc as plsc

N = 10000
E = 320000
D_IN = 128
HID = 128
K = 64
OUT = 64

NC = 2
NS = 16
NW = NC * NS
CHUNK = 125
NCHUNK = E // (NW * CHUNK)
NBUF = 4
N_PAD = 10240
RPT = N_PAD // NS
STG = 320


def _stage_indices_and_zero(ei4, row_v, col_v, znk, acc, w, sid):
    pltpu.sync_copy(ei4.at[0, w], row_v)
    pltpu.sync_copy(ei4.at[1, w], col_v)
    sl = pl.ds(sid * RPT, RPT)
    pltpu.sync_copy(znk.at[sl], acc.at[sl])
    return sl


def _scatter_pipeline(tab, row_v, col_v, vals, acc, gsem, ssem, deg_args):
    for b in range(NBUF):
        pltpu.async_copy(tab.at[row_v.at[b]], vals.at[b], gsem.at[b])

    def quad(i, carry):
        for b in range(NBUF):
            j = NBUF * i + b
            pltpu.make_async_copy(tab.at[row_v.at[j]], vals.at[b],
                                  gsem.at[b]).wait()
            pltpu.async_copy(vals.at[b], acc.at[col_v.at[j]], ssem.at[b],
                             add=True)
            if deg_args is not None:
                ones_v, deg, dsem = deg_args
                pltpu.async_copy(ones_v, deg.at[col_v.at[j]], dsem,
                                 add=True)

            @pl.when(j + NBUF < NCHUNK)
            def _():
                pltpu.make_async_copy(vals.at[b], acc.at[col_v.at[j]],
                                      ssem.at[b]).wait()
                pltpu.async_copy(tab.at[row_v.at[j + NBUF]], vals.at[b],
                                 gsem.at[b])
            if deg_args is not None:
                ones_v, deg, dsem = deg_args

                @pl.when(j >= NBUF)
                def _():
                    pltpu.make_async_copy(ones_v, deg.at[col_v.at[j]],
                                          dsem).wait()
        return carry

    lax.fori_loop(0, NCHUNK // NBUF, quad, 0)

    for b in range(NBUF):
        j = NCHUNK - NBUF + b
        pltpu.make_async_copy(vals.at[b], acc.at[col_v.at[j]],
                              ssem.at[b]).wait()
        if deg_args is not None:
            ones_v, deg, dsem = deg_args
            pltpu.make_async_copy(ones_v, deg.at[col_v.at[j]], dsem).wait()


def _segsum_deg_body(tab, ei4, znk, zn, ones_h, out, outdeg,
                     row_v, col_v, vals, ones_v, acc, deg, gsem, ssem, dsem):
    cid = lax.axis_index("c")
    sid = lax.axis_index("s")
    w = cid * NS + sid
    sl = _stage_indices_and_zero(ei4, row_v, col_v, znk, acc, w, sid)
    pltpu.sync_copy(ones_h, ones_v)

    @pl.when(sid == 0)
    def _():
        pltpu.sync_copy(zn, deg)
    plsc.subcore_barrier()

    _scatter_pipeline(tab, row_v, col_v, vals, acc, gsem, ssem,
                      (ones_v, deg, dsem))

    plsc.subcore_barrier()
    pltpu.sync_copy(acc.at[sl], out.at[cid, sl])

    @pl.when(sid == 0)
    def _():
        pltpu.sync_copy(deg, outdeg.at[cid])


def _segsum_num_body(tab, ei4, znk, outnum,
                     row_v, col_v, vals, num_v, acc, gsem, ssem):
    cid = lax.axis_index("c")
    sid = lax.axis_index("s")
    w = cid * NS + sid
    _stage_indices_and_zero(ei4, row_v, col_v, znk, acc, w, sid)
    plsc.subcore_barrier()

    _scatter_pipeline(tab, row_v, col_v, vals, acc, gsem, ssem, None)

    plsc.subcore_barrier()

    base = sid * RPT
    total = jnp.zeros((16,), jnp.float32)
    off = 0
    for sz in [120, 120, 120, 120, 120, 40]:
        pltpu.sync_copy(acc.at[pl.ds(base + off, sz)],
                        vals.at[0, pl.ds(0, sz)])
        pltpu.sync_copy(tab.at[pl.ds(base + off, sz)],
                        vals.at[1, pl.ds(0, sz)])

        def rowbody(r, c):
            c0, c1, c2, c3 = c
            c0 = c0 + vals[0, r, pl.ds(0, 16)] * vals[1, r, pl.ds(0, 16)]
            c1 = c1 + vals[0, r, pl.ds(16, 16)] * vals[1, r, pl.ds(16, 16)]
            c2 = c2 + vals[0, r, pl.ds(32, 16)] * vals[1, r, pl.ds(32, 16)]
            c3 = c3 + vals[0, r, pl.ds(48, 16)] * vals[1, r, pl.ds(48, 16)]
            return (c0, c1, c2, c3)

        z16 = jnp.zeros((16,), jnp.float32)
        c0, c1, c2, c3 = lax.fori_loop(0, sz, rowbody, (z16, z16, z16, z16))
        total = total + c0 + c1 + c2 + c3
        off += sz
    num_v[...] = total
    pltpu.sync_copy(num_v, outnum.at[cid, sid])


@functools.cache
def _make_segsum_deg():
    mesh = plsc.VectorSubcoreMesh(core_axis_name="c", subcore_axis_name="s",
                                  num_cores=NC, num_subcores=NS)
    return pl.kernel(
        _segsum_deg_body,
        out_type=[jax.ShapeDtypeStruct((NC, N_PAD, K), jnp.float32),
                  jax.ShapeDtypeStruct((NC, N_PAD, 16), jnp.float32)],
        mesh=mesh,
        scratch_types=[
            pltpu.VMEM((NCHUNK, CHUNK), jnp.int32),
            pltpu.VMEM((NCHUNK, CHUNK), jnp.int32),
            pltpu.VMEM((NBUF, CHUNK, K), jnp.float32),
            pltpu.VMEM((CHUNK, 16), jnp.float32),
            pltpu.VMEM_SHARED((N_PAD, K), jnp.float32),
            pltpu.VMEM_SHARED((N_PAD, 16), jnp.float32),
            pltpu.SemaphoreType.DMA((NBUF,)),
            pltpu.SemaphoreType.DMA((NBUF,)),
            pltpu.SemaphoreType.DMA,
        ],
        compiler_params=pltpu.CompilerParams(use_tc_tiling_on_sc=False))


@functools.cache
def _make_segsum_num():
    mesh = plsc.VectorSubcoreMesh(core_axis_name="c", subcore_axis_name="s",
                                  num_cores=NC, num_subcores=NS)
    return pl.kernel(
        _segsum_num_body,
        out_type=jax.ShapeDtypeStruct((NC, NS, 16), jnp.float32),
        mesh=mesh,
        scratch_types=[
            pltpu.VMEM((NCHUNK, CHUNK), jnp.int32),
            pltpu.VMEM((NCHUNK, CHUNK), jnp.int32),
            pltpu.VMEM((NBUF, CHUNK, K), jnp.float32),
            pltpu.VMEM((16,), jnp.float32),
            pltpu.VMEM_SHARED((N_PAD, K), jnp.float32),
            pltpu.SemaphoreType.DMA((NBUF,)),
            pltpu.SemaphoreType.DMA((NBUF,)),
        ],
        compiler_params=pltpu.CompilerParams(use_tc_tiling_on_sc=False))


BM = 2000
BMS = 640


def _softmax_body(p, beig, s_ref):
    xa = p[0] + p[1] + beig[...]
    m = jnp.max(xa, axis=1, keepdims=True)
    e = jnp.exp(xa - m)
    s_ref[...] = e / jnp.sum(e, axis=1, keepdims=True)


def _mm_body(x, wx, bx, xx_ref):
    xx_ref[...] = (jnp.dot(x[...], wx[...], preferred_element_type=jnp.float32)
                   + bx[...])


def _dense_body(s_in, xx, dg, wf, bf, z_ref, ss_ref, den_ref, ss_acc,
                den_acc):
    i = pl.program_id(0)

    @pl.when(i == 0)
    def _():
        ss_acc[...] = jnp.zeros_like(ss_acc)
        den_acc[0, 0] = 0.0

    sb = s_in[...]
    deg = dg[0][:, 0:1] + dg[1][:, 0:1]
    ssq = jnp.sum(sb * sb, axis=1, keepdims=True)
    den_acc[0, 0] += jnp.sum(deg * ssq)
    ss_acc[...] += lax.dot_general(sb, sb, (((0,), (0,)), ((), ())),
                                   preferred_element_type=jnp.float32)

    z = (jnp.dot(xx[...], wf[:HID, :], preferred_element_type=jnp.float32)
         + jnp.dot(sb, wf[HID:, :], preferred_element_type=jnp.float32)
         + bf[...])
    m = jnp.max(z, axis=1, keepdims=True)
    z_ref[...] = z - m - jnp.log(jnp.sum(jnp.exp(z - m), axis=1,
                                         keepdims=True))

    @pl.when(i == pl.num_programs(0) - 1)
    def _():
        ss_ref[...] = ss_acc[...]
        den_ref[...] = jnp.full((1, 1), den_acc[0, 0], jnp.float32)


def _loss_body(ss, den, nump, loss_ref):
    SS = ss[...]
    fro = jnp.sqrt(jnp.sum(SS * SS))
    r = lax.broadcasted_iota(jnp.int32, (K, K), 0)
    c = lax.broadcasted_iota(jnp.int32, (K, K), 1)
    eye = jnp.where(r == c, 1.0, 0.0).astype(jnp.float32)
    T = SS / (fro + 1e-10) - eye / jnp.sqrt(jnp.float32(K))
    ortho = jnp.sqrt(jnp.sum(T * T))
    num = jnp.sum(nump[...])
    pump = -(num / (jnp.sum(den[...]) + 1e-10))
    loss_ref[...] = jnp.full((1, 1), pump + ortho, jnp.float32)


def kernel(x, edge_index, W_eig, b_eig, W_x, b_x, W_f, b_f):
    ei4 = edge_index.reshape(2, NW, NCHUNK, CHUNK)
    znk = jnp.zeros((N_PAD, K), jnp.float32)
    zn = jnp.zeros((N_PAD, 16), jnp.float32)
    ones_c = jnp.zeros((CHUNK, 16), jnp.float32).at[:, 0].set(1.0)

    grid = (N // BM,)

    xx = pl.pallas_call(
        _mm_body,
        grid=grid,
        in_specs=[
            pl.BlockSpec((BM, D_IN), lambda i: (i, 0)),
            pl.BlockSpec((D_IN, HID), lambda i: (0, 0)),
            pl.BlockSpec((1, HID), lambda i: (0, 0)),
        ],
        out_specs=pl.BlockSpec((BM, HID), lambda i: (i, 0)),
        out_shape=jax.ShapeDtypeStruct((N, HID), jnp.float32),
    )(x, W_x, b_x.reshape(1, HID))

    xap, degp = _make_segsum_deg()(W_eig, ei4, znk, zn, ones_c)

    s = pl.pallas_call(
        _softmax_body,
        grid=(N_PAD // BMS,),
        in_specs=[
            pl.BlockSpec((NC, BMS, K), lambda i: (0, i, 0)),
            pl.BlockSpec((1, K), lambda i: (0, 0)),
        ],
        out_specs=pl.BlockSpec((BMS, K), lambda i: (i, 0)),
        out_shape=jax.ShapeDtypeStruct((N_PAD, K), jnp.float32),
    )(xap, b_eig.reshape(1, K))

    nump = _make_segsum_num()(s, ei4, znk)

    z, ss, den = pl.pallas_call(
        _dense_body,
        grid=grid,
        in_specs=[
            pl.BlockSpec((BM, K), lambda i: (i, 0)),
            pl.BlockSpec((BM, HID), lambda i: (i, 0)),
            pl.BlockSpec((NC, BM, 16), lambda i: (0, i, 0)),
            pl.BlockSpec((HID + K, OUT), lambda i: (0, 0)),
            pl.BlockSpec((1, OUT), lambda i: (0, 0)),
        ],
        out_specs=[
            pl.BlockSpec((BM, OUT), lambda i: (i, 0)),
            pl.BlockSpec((K, K), lambda i: (0, 0)),
            pl.BlockSpec((1, 1), lambda i: (0, 0)),
        ],
        out_shape=[
            jax.ShapeDtypeStruct((N, OUT), jnp.float32),
            jax.ShapeDtypeStruct((K, K), jnp.float32),
            jax.ShapeDtypeStruct((1, 1), jnp.float32),
        ],
        scratch_shapes=[
            pltpu.VMEM((K, K), jnp.float32),
            pltpu.SMEM((1, 1), jnp.float32),
        ],
    )(s, xx, degp, W_f, b_f.reshape(1, OUT))

    loss = pl.pallas_call(
        _loss_body,
        grid=(1,),
        in_specs=[
            pl.BlockSpec((K, K), lambda i: (0, 0)),
            pl.BlockSpec((1, 1), lambda i: (0, 0)),
            pl.BlockSpec((NC, NS, 16), lambda i: (0, 0, 0)),
        ],
        out_specs=pl.BlockSpec((1, 1), lambda i: (0, 0)),
        out_shape=jax.ShapeDtypeStruct((1, 1), jnp.float32),
    )(ss, den, nump)

    return z, loss[0, 0]


# speedup vs baseline: 1.3536x; 1.0255x over previous
import functools

import jax
import jax.numpy as jnp
from jax import lax
from jax.experimental import pallas as pl
from jax.experimental.pallas import tpu as pltpu
from jax.experimental.pallas import tpu_sc as plsc

N = 10000
E = 320000
D_IN = 128
HID = 128
K = 64
OUT = 64

NC = 2
NS = 16
NW = NC * NS
CHUNK = 125
NCHUNK = E // (NW * CHUNK)
NBUF = 4
N_PAD = 10240
RPT = N_PAD // NS
STG = 320


def _stage_indices_and_zero(ei4, row_v, col_v, znk, acc, w, sid):
    pltpu.sync_copy(ei4.at[0, w], row_v)
    pltpu.sync_copy(ei4.at[1, w], col_v)
    sl = pl.ds(sid * RPT, RPT)
    pltpu.sync_copy(znk.at[sl], acc.at[sl])
    return sl


def _scatter_pipeline(tab, row_v, col_v, vals, acc, gsem, ssem, deg_args):
    for b in range(NBUF):
        pltpu.async_copy(tab.at[row_v.at[b]], vals.at[b], gsem.at[b])

    def quad(i, carry):
        for b in range(NBUF):
            j = NBUF * i + b
            pltpu.make_async_copy(tab.at[row_v.at[j]], vals.at[b],
                                  gsem.at[b]).wait()
            pltpu.async_copy(vals.at[b], acc.at[col_v.at[j]], ssem.at[b],
                             add=True)
            if deg_args is not None:
                ones_v, deg, dsem = deg_args
                pltpu.async_copy(ones_v, deg.at[col_v.at[j]], dsem,
                                 add=True)

            @pl.when(j + NBUF < NCHUNK)
            def _():
                pltpu.make_async_copy(vals.at[b], acc.at[col_v.at[j]],
                                      ssem.at[b]).wait()
                pltpu.async_copy(tab.at[row_v.at[j + NBUF]], vals.at[b],
                                 gsem.at[b])
            if deg_args is not None:
                ones_v, deg, dsem = deg_args

                @pl.when(j >= NBUF)
                def _():
                    pltpu.make_async_copy(ones_v, deg.at[col_v.at[j]],
                                          dsem).wait()
        return carry

    lax.fori_loop(0, NCHUNK // NBUF, quad, 0)

    for b in range(NBUF):
        j = NCHUNK - NBUF + b
        pltpu.make_async_copy(vals.at[b], acc.at[col_v.at[j]],
                              ssem.at[b]).wait()
        if deg_args is not None:
            ones_v, deg, dsem = deg_args
            pltpu.make_async_copy(ones_v, deg.at[col_v.at[j]], dsem).wait()


def _segsum_deg_body(tab, ei4, znk, zn, ones_h, out, outdeg,
                     row_v, col_v, vals, ones_v, acc, deg, gsem, ssem, dsem):
    cid = lax.axis_index("c")
    sid = lax.axis_index("s")
    w = cid * NS + sid
    sl = _stage_indices_and_zero(ei4, row_v, col_v, znk, acc, w, sid)
    pltpu.sync_copy(ones_h, ones_v)

    @pl.when(sid == 0)
    def _():
        pltpu.sync_copy(zn, deg)
    plsc.subcore_barrier()

    _scatter_pipeline(tab, row_v, col_v, vals, acc, gsem, ssem,
                      (ones_v, deg, dsem))

    plsc.subcore_barrier()
    pltpu.sync_copy(acc.at[sl], out.at[cid, sl])

    @pl.when(sid == 0)
    def _():
        pltpu.sync_copy(deg, outdeg.at[cid])


def _segsum_num_body(tab, ei4, znk, outnum,
                     row_v, col_v, vals, num_v, acc, gsem, ssem):
    cid = lax.axis_index("c")
    sid = lax.axis_index("s")
    w = cid * NS + sid
    _stage_indices_and_zero(ei4, row_v, col_v, znk, acc, w, sid)
    plsc.subcore_barrier()

    _scatter_pipeline(tab, row_v, col_v, vals, acc, gsem, ssem, None)

    plsc.subcore_barrier()

    base = sid * RPT
    total = jnp.zeros((16,), jnp.float32)
    off = 0
    for sz in [120, 120, 120, 120, 120, 40]:
        pltpu.sync_copy(acc.at[pl.ds(base + off, sz)],
                        vals.at[0, pl.ds(0, sz)])
        pltpu.sync_copy(tab.at[pl.ds(base + off, sz)],
                        vals.at[1, pl.ds(0, sz)])

        def rowbody(q, c):
            c0, c1, c2, c3 = c
            for u in range(4):
                r = 4 * q + u
                c0 = c0 + vals[0, r, pl.ds(0, 16)] * vals[1, r, pl.ds(0, 16)]
                c1 = c1 + vals[0, r, pl.ds(16, 16)] * vals[1, r, pl.ds(16, 16)]
                c2 = c2 + vals[0, r, pl.ds(32, 16)] * vals[1, r, pl.ds(32, 16)]
                c3 = c3 + vals[0, r, pl.ds(48, 16)] * vals[1, r, pl.ds(48, 16)]
            return (c0, c1, c2, c3)

        z16 = jnp.zeros((16,), jnp.float32)
        c0, c1, c2, c3 = lax.fori_loop(0, sz // 4, rowbody,
                                       (z16, z16, z16, z16))
        total = total + c0 + c1 + c2 + c3
        off += sz
    num_v[...] = total
    pltpu.sync_copy(num_v, outnum.at[cid, sid])


@functools.cache
def _make_segsum_deg():
    mesh = plsc.VectorSubcoreMesh(core_axis_name="c", subcore_axis_name="s",
                                  num_cores=NC, num_subcores=NS)
    return pl.kernel(
        _segsum_deg_body,
        out_type=[jax.ShapeDtypeStruct((NC, N_PAD, K), jnp.float32),
                  jax.ShapeDtypeStruct((NC, N_PAD, 16), jnp.float32)],
        mesh=mesh,
        scratch_types=[
            pltpu.VMEM((NCHUNK, CHUNK), jnp.int32),
            pltpu.VMEM((NCHUNK, CHUNK), jnp.int32),
            pltpu.VMEM((NBUF, CHUNK, K), jnp.float32),
            pltpu.VMEM((CHUNK, 16), jnp.float32),
            pltpu.VMEM_SHARED((N_PAD, K), jnp.float32),
            pltpu.VMEM_SHARED((N_PAD, 16), jnp.float32),
            pltpu.SemaphoreType.DMA((NBUF,)),
            pltpu.SemaphoreType.DMA((NBUF,)),
            pltpu.SemaphoreType.DMA,
        ],
        compiler_params=pltpu.CompilerParams(use_tc_tiling_on_sc=False))


@functools.cache
def _make_segsum_num():
    mesh = plsc.VectorSubcoreMesh(core_axis_name="c", subcore_axis_name="s",
                                  num_cores=NC, num_subcores=NS)
    return pl.kernel(
        _segsum_num_body,
        out_type=jax.ShapeDtypeStruct((NC, NS, 16), jnp.float32),
        mesh=mesh,
        scratch_types=[
            pltpu.VMEM((NCHUNK, CHUNK), jnp.int32),
            pltpu.VMEM((NCHUNK, CHUNK), jnp.int32),
            pltpu.VMEM((NBUF, CHUNK, K), jnp.float32),
            pltpu.VMEM((16,), jnp.float32),
            pltpu.VMEM_SHARED((N_PAD, K), jnp.float32),
            pltpu.SemaphoreType.DMA((NBUF,)),
            pltpu.SemaphoreType.DMA((NBUF,)),
        ],
        compiler_params=pltpu.CompilerParams(use_tc_tiling_on_sc=False))


BM = 2000
BMS = 2048


def _softmax_body(p, beig, s_ref):
    xa = p[0] + p[1] + beig[...]
    m = jnp.max(xa, axis=1, keepdims=True)
    e = jnp.exp(xa - m)
    s_ref[...] = e / jnp.sum(e, axis=1, keepdims=True)


def _mm_body(x, wx, bx, xx_ref):
    xx_ref[...] = (jnp.dot(x[...], wx[...], preferred_element_type=jnp.float32)
                   + bx[...])


def _dense_body(s_in, xx, dg, wf, bf, z_ref, ss_ref, den_ref, ss_acc,
                den_acc):
    i = pl.program_id(0)

    @pl.when(i == 0)
    def _():
        ss_acc[...] = jnp.zeros_like(ss_acc)
        den_acc[0, 0] = 0.0

    sb = s_in[...]
    deg = dg[0][:, 0:1] + dg[1][:, 0:1]
    ssq = jnp.sum(sb * sb, axis=1, keepdims=True)
    den_acc[0, 0] += jnp.sum(deg * ssq)
    ss_acc[...] += lax.dot_general(sb, sb, (((0,), (0,)), ((), ())),
                                   preferred_element_type=jnp.float32)

    z = (jnp.dot(xx[...], wf[:HID, :], preferred_element_type=jnp.float32)
         + jnp.dot(sb, wf[HID:, :], preferred_element_type=jnp.float32)
         + bf[...])
    m = jnp.max(z, axis=1, keepdims=True)
    z_ref[...] = z - m - jnp.log(jnp.sum(jnp.exp(z - m), axis=1,
                                         keepdims=True))

    @pl.when(i == pl.num_programs(0) - 1)
    def _():
        ss_ref[...] = ss_acc[...]
        den_ref[...] = jnp.full((1, 1), den_acc[0, 0], jnp.float32)


def _loss_body(ss, den, nump, loss_ref):
    SS = ss[...]
    fro = jnp.sqrt(jnp.sum(SS * SS))
    r = lax.broadcasted_iota(jnp.int32, (K, K), 0)
    c = lax.broadcasted_iota(jnp.int32, (K, K), 1)
    eye = jnp.where(r == c, 1.0, 0.0).astype(jnp.float32)
    T = SS / (fro + 1e-10) - eye / jnp.sqrt(jnp.float32(K))
    ortho = jnp.sqrt(jnp.sum(T * T))
    num = jnp.sum(nump[...])
    pump = -(num / (jnp.sum(den[...]) + 1e-10))
    loss_ref[...] = jnp.full((1, 1), pump + ortho, jnp.float32)


def kernel(x, edge_index, W_eig, b_eig, W_x, b_x, W_f, b_f):
    ei4 = edge_index.reshape(2, NW, NCHUNK, CHUNK)
    znk = jnp.zeros((N_PAD, K), jnp.float32)
    zn = jnp.zeros((N_PAD, 16), jnp.float32)
    ones_c = jnp.zeros((CHUNK, 16), jnp.float32).at[:, 0].set(1.0)

    grid = (N // BM,)

    xx = pl.pallas_call(
        _mm_body,
        grid=grid,
        in_specs=[
            pl.BlockSpec((BM, D_IN), lambda i: (i, 0)),
            pl.BlockSpec((D_IN, HID), lambda i: (0, 0)),
            pl.BlockSpec((1, HID), lambda i: (0, 0)),
        ],
        out_specs=pl.BlockSpec((BM, HID), lambda i: (i, 0)),
        out_shape=jax.ShapeDtypeStruct((N, HID), jnp.float32),
    )(x, W_x, b_x.reshape(1, HID))

    xap, degp = _make_segsum_deg()(W_eig, ei4, znk, zn, ones_c)

    s = pl.pallas_call(
        _softmax_body,
        grid=(N_PAD // BMS,),
        in_specs=[
            pl.BlockSpec((NC, BMS, K), lambda i: (0, i, 0)),
            pl.BlockSpec((1, K), lambda i: (0, 0)),
        ],
        out_specs=pl.BlockSpec((BMS, K), lambda i: (i, 0)),
        out_shape=jax.ShapeDtypeStruct((N_PAD, K), jnp.float32),
    )(xap, b_eig.reshape(1, K))

    nump = _make_segsum_num()(s, ei4, znk)

    z, ss, den = pl.pallas_call(
        _dense_body,
        grid=grid,
        in_specs=[
            pl.BlockSpec((BM, K), lambda i: (i, 0)),
            pl.BlockSpec((BM, HID), lambda i: (i, 0)),
            pl.BlockSpec((NC, BM, 16), lambda i: (0, i, 0)),
            pl.BlockSpec((HID + K, OUT), lambda i: (0, 0)),
            pl.BlockSpec((1, OUT), lambda i: (0, 0)),
        ],
        out_specs=[
            pl.BlockSpec((BM, OUT), lambda i: (i, 0)),
            pl.BlockSpec((K, K), lambda i: (0, 0)),
            pl.BlockSpec((1, 1), lambda i: (0, 0)),
        ],
        out_shape=[
            jax.ShapeDtypeStruct((N, OUT), jnp.float32),
            jax.ShapeDtypeStruct((K, K), jnp.float32),
            jax.ShapeDtypeStruct((1, 1), jnp.float32),
        ],
        scratch_shapes=[
            pltpu.VMEM((K, K), jnp.float32),
            pltpu.SMEM((1, 1), jnp.float32),
        ],
    )(s, xx, degp, W_f, b_f.reshape(1, OUT))

    loss = pl.pallas_call(
        _loss_body,
        grid=(1,),
        in_specs=[
            pl.BlockSpec((K, K), lambda i: (0, 0)),
            pl.BlockSpec((1, 1), lambda i: (0, 0)),
            pl.BlockSpec((NC, NS, 16), lambda i: (0, 0, 0)),
        ],
        out_specs=pl.BlockSpec((1, 1), lambda i: (0, 0)),
        out_shape=jax.ShapeDtypeStruct((1, 1), jnp.float32),
    )(ss, den, nump)

    return z, loss[0, 0]
